# Initial kernel scaffold; baseline (speedup 1.0000x reference)
import jax
import jax.numpy as jnp
from jax.experimental import pallas as pl
from jax.experimental.pallas import tpu as pltpu
from jax.experimental.pallas import tpu_sc as plsc

BATCH = 8192
NUM_NEURONS = 2500
INPUT_DIM = 128
GRID_W = 50

BLK_B = 256
NUM_BLKS = BATCH // BLK_B

GATHER_WINDOW = 128


def _bmu_kernel(x_ref, w_ref, bmu_ref, fi_ref, fj_ref):
    x = x_ref[...]
    w = w_ref[...]
    xw = jax.lax.dot_general(
        x, w, (((1,), (1,)), ((), ())),
        preferred_element_type=jnp.float32,
        precision=jax.lax.Precision.HIGHEST,
    )
    x2 = jnp.sum(x * x, axis=1, keepdims=True)
    w2 = jnp.sum(w * w, axis=1)
    d2 = (x2 - 2.0 * xw) + w2[None, :]
    bmu = jnp.argmin(d2, axis=1).astype(jnp.int32)
    bmu_ref[0, 0, :] = bmu
    fi_ref[0, 0, :] = (bmu // GRID_W).astype(jnp.float32)
    fj_ref[0, 0, :] = (bmu % GRID_W).astype(jnp.float32)


def _compute_bmu(x, weights):
    out_types = (
        jax.ShapeDtypeStruct((NUM_BLKS, 1, BLK_B), jnp.int32),
        jax.ShapeDtypeStruct((NUM_BLKS, 1, BLK_B), jnp.float32),
        jax.ShapeDtypeStruct((NUM_BLKS, 1, BLK_B), jnp.float32),
    )
    bmu, fi, fj = pl.pallas_call(
        _bmu_kernel,
        grid=(NUM_BLKS,),
        in_specs=[
            pl.BlockSpec((BLK_B, INPUT_DIM), lambda i: (i, 0)),
            pl.BlockSpec((NUM_NEURONS, INPUT_DIM), lambda i: (0, 0)),
        ],
        out_specs=(
            pl.BlockSpec((1, 1, BLK_B), lambda i: (i, 0, 0)),
            pl.BlockSpec((1, 1, BLK_B), lambda i: (i, 0, 0)),
            pl.BlockSpec((1, 1, BLK_B), lambda i: (i, 0, 0)),
        ),
        out_shape=out_types,
    )(x, weights)
    return bmu.reshape(BATCH), fi.reshape(BATCH), fj.reshape(BATCH)


def _sc_gather(weights, indices):
    indices = indices.reshape((1, BATCH))
    mesh = plsc.VectorSubcoreMesh(core_axis_name="core",
                                  subcore_axis_name="subcore")

    @pl.kernel(
        out_type=jax.ShapeDtypeStruct((BATCH, INPUT_DIM), weights.dtype),
        mesh=mesh,
    )
    def gather_kernel(w_hbm, i_hbm, o_hbm):
        def body(i_vmem, o_vmem):
            pltpu.sync_copy(w_hbm.at[i_vmem.at[0]], o_vmem)

        pltpu.emit_pipeline(
            body,
            grid=(BATCH // GATHER_WINDOW,),
            in_specs=[pl.BlockSpec((1, GATHER_WINDOW),
                                   index_map=lambda i: (0, i))],
            out_specs=[pl.BlockSpec((GATHER_WINDOW, INPUT_DIM),
                                    index_map=lambda i: (i, 0))],
            core_axis_name=("core", "subcore"),
            dimension_semantics=(pltpu.PARALLEL,),
        )(i_hbm, o_hbm)

    return gather_kernel(weights, indices)


def kernel(x, weights):
    bmu, fi, fj = _compute_bmu(x, weights)
    quantized = _sc_gather(weights, bmu)
    bmu_locs = jnp.stack([fi, fj], axis=1)
    return quantized, bmu_locs


# speedup vs baseline: 1.0137x; 1.0137x over previous
import jax
import jax.numpy as jnp
from jax.experimental import pallas as pl
from jax.experimental.pallas import tpu as pltpu
from jax.experimental.pallas import tpu_sc as plsc

BATCH = 8192
NUM_NEURONS = 2500
INPUT_DIM = 128
GRID_W = 50

BLK_B = 256
NUM_BLKS = BATCH // BLK_B

GATHER_WINDOW = 128


def _bmu_kernel(x_ref, w_ref, bmu_ref, fi_ref, fj_ref):
    x = x_ref[...]
    w = w_ref[...]
    xw = jax.lax.dot_general(
        x.astype(jnp.bfloat16), w.astype(jnp.bfloat16),
        (((1,), (1,)), ((), ())),
        preferred_element_type=jnp.float32,
    )
    x2 = jnp.sum(x * x, axis=1, keepdims=True)
    w2 = jnp.sum(w * w, axis=1)
    d2 = (x2 - 2.0 * xw) + w2[None, :]
    bmu = jnp.argmin(d2, axis=1).astype(jnp.int32)
    bmu_ref[0, 0, :] = bmu
    fi_ref[0, 0, :] = (bmu // GRID_W).astype(jnp.float32)
    fj_ref[0, 0, :] = (bmu % GRID_W).astype(jnp.float32)


def _compute_bmu(x, weights):
    out_types = (
        jax.ShapeDtypeStruct((NUM_BLKS, 1, BLK_B), jnp.int32),
        jax.ShapeDtypeStruct((NUM_BLKS, 1, BLK_B), jnp.float32),
        jax.ShapeDtypeStruct((NUM_BLKS, 1, BLK_B), jnp.float32),
    )
    bmu, fi, fj = pl.pallas_call(
        _bmu_kernel,
        grid=(NUM_BLKS,),
        in_specs=[
            pl.BlockSpec((BLK_B, INPUT_DIM), lambda i: (i, 0)),
            pl.BlockSpec((NUM_NEURONS, INPUT_DIM), lambda i: (0, 0)),
        ],
        out_specs=(
            pl.BlockSpec((1, 1, BLK_B), lambda i: (i, 0, 0)),
            pl.BlockSpec((1, 1, BLK_B), lambda i: (i, 0, 0)),
            pl.BlockSpec((1, 1, BLK_B), lambda i: (i, 0, 0)),
        ),
        out_shape=out_types,
    )(x, weights)
    return bmu.reshape(BATCH), fi.reshape(BATCH), fj.reshape(BATCH)


def _sc_gather(weights, indices):
    indices = indices.reshape((1, BATCH))
    mesh = plsc.VectorSubcoreMesh(core_axis_name="core",
                                  subcore_axis_name="subcore")

    @pl.kernel(
        out_type=jax.ShapeDtypeStruct((BATCH, INPUT_DIM), weights.dtype),
        mesh=mesh,
    )
    def gather_kernel(w_hbm, i_hbm, o_hbm):
        def body(i_vmem, o_vmem):
            pltpu.sync_copy(w_hbm.at[i_vmem.at[0]], o_vmem)

        pltpu.emit_pipeline(
            body,
            grid=(BATCH // GATHER_WINDOW,),
            in_specs=[pl.BlockSpec((1, GATHER_WINDOW),
                                   index_map=lambda i: (0, i))],
            out_specs=[pl.BlockSpec((GATHER_WINDOW, INPUT_DIM),
                                    index_map=lambda i: (i, 0))],
            core_axis_name=("core", "subcore"),
            dimension_semantics=(pltpu.PARALLEL,),
        )(i_hbm, o_hbm)

    return gather_kernel(weights, indices)


def kernel(x, weights):
    bmu, fi, fj = _compute_bmu(x, weights)
    quantized = _sc_gather(weights, bmu)
    bmu_locs = jnp.stack([fi, fj], axis=1)
    return quantized, bmu_locs
